# Initial kernel scaffold; baseline (speedup 1.0000x reference)
import jax
import jax.numpy as jnp
from jax.experimental import pallas as pl


def kernel(x, W0, b0, W1, b1, W2, b2, W3, b3, W4, b4):
    raise NotImplementedError("write your pallas kernel here")


# speedup vs baseline: 203.3975x; 203.3975x over previous
import numpy as np
import jax
import jax.numpy as jnp
from jax.experimental import pallas as pl

_PAR = [-1, 0, 0, 0, 0, 1, 1, 2, 3, 4, 4, 5, 5, 6, 7, 8, 9, 10,
        11, 12, 13, 14, 14, 14, 15, 15, 16, 17]
_N = 28


def _build_tree_consts():
    ch = [[] for _ in range(_N)]
    for c, p in enumerate(_PAR):
        if p >= 0:
            ch[p].append(c)
    deg = np.zeros(_N, dtype=np.float64)
    for n in range(_N):
        deg[n] = 1.0 + len(ch[n]) + (1.0 if _PAR[n] >= 0 else 0.0)
    dis = 1.0 / np.sqrt(deg)
    rows = []
    for n in range(_N):
        nbrs = [n] + ([_PAR[n]] if _PAR[n] >= 0 else []) + ch[n]
        rows.append([(m, float(dis[n] * dis[m])) for m in nbrs])
    s1 = np.array([sum(w for _, w in r) for r in rows], dtype=np.float32)
    return rows, s1


_A_ROWS, _S1 = _build_tree_consts()

_PARENT_L2 = [0, 0, 1, 2, 3, 3]
_PARENT_L3 = [0, 0, 1, 2, 3, 4, 5]
_PARENT_L4 = [0, 1, 2, 3, 3, 3, 4, 4, 5, 6]


def _gather_cols(arr, idx):
    return jnp.concatenate([arr[:, j:j + 1] for j in idx], axis=1)

_LAYER_DIMS = [(64, 32), (32, 16), (16, 8)]


def _group_softmax(l, a, b):
    seg = l[:, a:b]
    m = jnp.max(seg, axis=1, keepdims=True)
    e = jnp.exp(seg - m)
    s = jnp.sum(e, axis=1, keepdims=True)
    return e / s


def _body(x_ref, w0_ref, b0_ref, w1_ref, b1_ref, w2_ref, b2_ref,
          w3_ref, b3_ref, w4_ref, b4_ref, out_ref):
    t = x_ref.shape[0]
    f32 = jnp.float32

    h0 = jnp.dot(x_ref[...], w0_ref[...], preferred_element_type=f32)
    b0 = b0_ref[...]
    nf = jnp.stack([jnp.maximum(h0 * float(_S1[n]) + b0, 0.0)
                    for n in range(_N)], axis=0)

    for (din, dout), wref, bref in zip(
            _LAYER_DIMS, (w1_ref, w2_ref, w3_ref), (b1_ref, b2_ref, b3_ref)):
        h = jnp.dot(nf.reshape(_N * t, din), wref[...],
                    preferred_element_type=f32).reshape(_N, t, dout)
        outs = []
        for n in range(_N):
            (m0, w0c), rest = _A_ROWS[n][0], _A_ROWS[n][1:]
            acc = h[m0] * w0c
            for m, w in rest:
                acc = acc + h[m] * w
            outs.append(acc)
        nf = jnp.maximum(jnp.stack(outs, axis=0) + bref[...][None, :, :], 0.0)

    w4 = w4_ref[...]
    cols = [jnp.sum(nf[n] * w4, axis=1, keepdims=True) for n in range(_N)]
    logits = jnp.concatenate(cols, axis=1) + b4_ref[...]

    ones = lambda k: jnp.ones((t, k), dtype=f32)
    c_14 = _group_softmax(logits, 1, 5)
    c_56 = _group_softmax(logits, 5, 7)
    c_910 = _group_softmax(logits, 9, 11)
    c_1112 = _group_softmax(logits, 11, 13)
    c_2123 = _group_softmax(logits, 21, 24)
    c_2425 = _group_softmax(logits, 24, 26)

    cond_l2 = jnp.concatenate([c_56, ones(2), c_910], axis=1)
    cond_l3 = jnp.concatenate([c_1112, ones(5)], axis=1)
    cond_l4 = jnp.concatenate([ones(3), c_2123, c_2425, ones(2)], axis=1)

    pp1 = c_14
    pp2 = cond_l2 * _gather_cols(pp1, _PARENT_L2)
    pp3 = cond_l3 * _gather_cols(pp2, _PARENT_L3)
    pp4 = cond_l4 * _gather_cols(pp3, _PARENT_L4)

    out_ref[...] = jnp.concatenate([ones(1), pp1, pp2, pp3, pp4], axis=1)


def kernel(x, W0, b0, W1, b1, W2, b2, W3, b3, W4, b4):
    batch, in_dim = x.shape
    t = 256
    grid = (batch // t,)
    const = lambda i: (0, 0)
    out = pl.pallas_call(
        _body,
        grid=grid,
        in_specs=[
            pl.BlockSpec((t, in_dim), lambda i: (i, 0)),
            pl.BlockSpec(W0.shape, const),
            pl.BlockSpec((1, 64), const),
            pl.BlockSpec(W1.shape, const),
            pl.BlockSpec((1, 32), const),
            pl.BlockSpec(W2.shape, const),
            pl.BlockSpec((1, 16), const),
            pl.BlockSpec(W3.shape, const),
            pl.BlockSpec((1, 8), const),
            pl.BlockSpec((1, 8), const),
            pl.BlockSpec((1, 1), const),
        ],
        out_specs=pl.BlockSpec((t, _N), lambda i: (i, 0)),
        out_shape=jax.ShapeDtypeStruct((batch, _N), jnp.float32),
    )(x, W0, b0.reshape(1, 64), W1, b1.reshape(1, 32), W2, b2.reshape(1, 16),
      W3, b3.reshape(1, 8), W4.reshape(1, 8), b4.reshape(1, 1))
    return out
